# Initial kernel scaffold; baseline (speedup 1.0000x reference)
import functools

import jax
import jax.numpy as jnp
from jax import lax
from jax.experimental import pallas as pl
from jax.experimental.pallas import tpu as pltpu
from jax.experimental.pallas import tpu_sc as plsc

N = 5
NE = 25

_XR = 0
_W1R = 5
_B1R = 25
_W2R = 29
_B2R = 157
_W3R = 159
_B3R = 161
_WLR = 162
_BLR = 172
_IR = 174
_ROWS = 178

_GDN = lax.GatherDimensionNumbers(
    offset_dims=(), collapsed_slice_dims=(0,), start_index_map=(0,))


def _gather16(v, idx):
    return lax.gather(v, idx[:, None], _GDN, (1,),
                      mode=lax.GatherScatterMode.PROMISE_IN_BOUNDS)


def _bcast(v, k):
    return _gather16(v, jnp.full((16,), k, dtype=jnp.int32))


def _rsqrt16(x):
    i = plsc.bitcast(x, jnp.int32)
    y = plsc.bitcast(jnp.int32(0x5F3759DF) - (i >> 1), jnp.float32)
    for _ in range(3):
        y = y * (1.5 - 0.5 * x * y * y)
    return y


@functools.partial(
    pl.kernel,
    out_type=jax.ShapeDtypeStruct((2, 16), jnp.float32),
    mesh=plsc.VectorSubcoreMesh(core_axis_name="c", subcore_axis_name="s"),
    scratch_types=[
        pltpu.VMEM((_ROWS, 16), jnp.float32),
        pltpu.VMEM((8, 16), jnp.float32),
        pltpu.VMEM((16,), jnp.float32),
        pltpu.VMEM((2, 16), jnp.float32),
    ],
)
def _sc_gcn(pack_hbm, out_hbm, fv, a_ref, deg_ref, out_ref):
    cid = lax.axis_index("c")
    sid = lax.axis_index("s")

    @pl.when(jnp.logical_and(cid == 0, sid == 0))
    def _body():
        pltpu.sync_copy(pack_hbm, fv)

        lane = lax.iota(jnp.int32, 16)
        one = jnp.ones((16,), jnp.float32)
        zero = jnp.zeros((16,), jnp.float32)

        src0 = plsc.bitcast(fv[_IR, :], jnp.int32)
        src1 = plsc.bitcast(fv[_IR + 1, :], jnp.int32)
        dst0 = plsc.bitcast(fv[_IR + 2, :], jnp.int32)
        dst1 = plsc.bitcast(fv[_IR + 3, :], jnp.int32)

        deg_ref[...] = zero
        for e in range(16):
            plsc.addupdate_scatter(deg_ref, [dst0], one, mask=lane == e)
        for e in range(NE - 16):
            plsc.addupdate_scatter(deg_ref, [dst1], one, mask=lane == e)
        dinv = _rsqrt16(jnp.maximum(deg_ref[...], 1.0))

        norm0 = _gather16(dinv, src0) * _gather16(dinv, dst0)
        norm1 = _gather16(dinv, src1) * _gather16(dinv, dst1)
        for i in range(N):
            a_ref[i, :] = zero
        for e in range(16):
            plsc.addupdate_scatter(a_ref, [dst0, src0], norm0, mask=lane == e)
        for e in range(NE - 16):
            plsc.addupdate_scatter(a_ref, [dst1, src1], norm1, mask=lane == e)
        a = [[_bcast(a_ref[i, :], j) for j in range(N)] for i in range(N)]

        xv = [fv[_XR + n, :] for n in range(N)]
        w1 = [[fv[_W1R + k * 4 + b, :] for b in range(4)] for k in range(N)]
        t1 = []
        for n in range(N):
            acc = [None] * 4
            for k in range(N):
                s = _bcast(xv[n], k)
                for b in range(4):
                    p = s * w1[k][b]
                    acc[b] = p if acc[b] is None else acc[b] + p
            t1.append(acc)
        b1v = [fv[_B1R + b, :] for b in range(4)]
        h1 = []
        for i in range(N):
            hb = []
            for b in range(4):
                g = None
                for j in range(N):
                    p = a[i][j] * t1[j][b]
                    g = p if g is None else g + p
                hb.append(jnp.maximum(g + b1v[b], 0.0))
            h1.append(hb)

        t2 = []
        for n in range(N):
            acc = [None, None]
            for k in range(64):
                s = _bcast(h1[n][k // 16], k % 16)
                for b in range(2):
                    p = s * fv[_W2R + k * 2 + b, :]
                    acc[b] = p if acc[b] is None else acc[b] + p
            t2.append(acc)
        b2v = [fv[_B2R + b, :] for b in range(2)]
        h2 = []
        for i in range(N):
            hb = []
            for b in range(2):
                g = None
                for j in range(N):
                    p = a[i][j] * t2[j][b]
                    g = p if g is None else g + p
                hb.append(jnp.maximum(g + b2v[b], 0.0))
            h2.append(hb)

        w3v = [fv[_W3R, :], fv[_W3R + 1, :]]
        u = []
        for n in range(N):
            d = h2[n][0] * w3v[0] + h2[n][1] * w3v[1]
            u.append(_bcast(plsc.cumsum(d), 15))
        b3v = fv[_B3R, :]
        vsp = []
        for i in range(N):
            g = None
            for j in range(N):
                p = a[i][j] * u[j]
                g = p if g is None else g + p
            vsp.append(g + b3v)

        o = []
        for b in range(2):
            g = fv[_BLR + b, :]
            for i in range(N):
                g = g + vsp[i] * fv[_WLR + i * 2 + b, :]
            o.append(g)
        out_ref[0, :] = o[0]
        out_ref[1, :] = o[1]
        pltpu.sync_copy(out_ref, out_hbm)


def kernel(x, edge_index, W1, b1, W2, b2, W3, b3, Wl, bl):
    ei = edge_index.astype(jnp.int32)
    loop = jnp.arange(N, dtype=jnp.int32)
    src = jnp.pad(jnp.concatenate([ei[0], loop]), (0, 32 - NE)).reshape(2, 16)
    dst = jnp.pad(jnp.concatenate([ei[1], loop]), (0, 32 - NE)).reshape(2, 16)
    idxf = lax.bitcast_convert_type(
        jnp.concatenate([src, dst], axis=0), jnp.float32)

    pack = jnp.concatenate([
        jnp.pad(x.astype(jnp.float32), ((0, 0), (0, 16 - x.shape[1]))),
        W1.reshape(20, 16),
        b1.reshape(4, 16),
        W2.reshape(128, 16),
        b2.reshape(2, 16),
        W3.reshape(-1).reshape(2, 16),
        jnp.broadcast_to(b3, (16,)).reshape(1, 16),
        jnp.pad(Wl.T, ((0, 0), (0, 7))).reshape(10, 16),
        jnp.pad(bl, (0, 7)).reshape(2, 16),
        idxf,
    ], axis=0)

    res = _sc_gcn(pack)
    return res.reshape(32)[:NE].reshape(N, N)


# speedup vs baseline: 1.1593x; 1.1593x over previous
import functools

import jax
import jax.numpy as jnp
from jax import lax
from jax.experimental import pallas as pl
from jax.experimental.pallas import tpu as pltpu
from jax.experimental.pallas import tpu_sc as plsc

N = 5
NE = 25

_XR = 0
_W1R = 5
_B1R = 25
_W2R = 29
_B2R = 157
_W3R = 159
_B3R = 161
_WLR = 162
_BLR = 172
_LUTR = 174
_ROWS = 176

_GDN = lax.GatherDimensionNumbers(
    offset_dims=(), collapsed_slice_dims=(0,), start_index_map=(0,))


def _gather16(v, idx):
    return lax.gather(v, idx[:, None], _GDN, (1,),
                      mode=lax.GatherScatterMode.PROMISE_IN_BOUNDS)


def _bcast(v, k):
    return _gather16(v, jnp.full((16,), k, dtype=jnp.int32))


def _bf16r(v):
    i = plsc.bitcast(v, jnp.int32)
    r = (i + jnp.int32(0x7FFF) + ((i >> 16) & 1)) & jnp.int32(-65536)
    return plsc.bitcast(r, jnp.float32)


@functools.partial(
    pl.kernel,
    out_type=jax.ShapeDtypeStruct((2, 16), jnp.float32),
    mesh=plsc.VectorSubcoreMesh(core_axis_name="c", subcore_axis_name="s"),
    compiler_params=pltpu.CompilerParams(needs_layout_passes=False),
    scratch_types=[
        pltpu.VMEM((_ROWS, 16), jnp.float32),
        pltpu.VMEM((4, 16), jnp.int32),
        pltpu.VMEM((8, 16), jnp.float32),
        pltpu.VMEM((16,), jnp.float32),
        pltpu.VMEM((2, 16), jnp.float32),
    ],
)
def _sc_gcn(pack_hbm, idx_hbm, out_hbm, fv, iv, a_ref, deg_ref, out_ref):
    cid = lax.axis_index("c")
    sid = lax.axis_index("s")

    @pl.when(jnp.logical_and(cid == 0, sid == 0))
    def _body():
        pltpu.sync_copy(pack_hbm, fv)
        pltpu.sync_copy(idx_hbm, iv)

        lane = lax.iota(jnp.int32, 16)
        one = jnp.ones((16,), jnp.float32)
        zero = jnp.zeros((16,), jnp.float32)

        src0 = iv[0, :]
        src1 = iv[1, :]
        dst0 = iv[2, :]
        dst1 = iv[3, :]

        deg_ref[...] = zero
        for e in range(16):
            plsc.addupdate_scatter(deg_ref, [dst0], one, mask=lane == e)
        for e in range(NE - 16):
            plsc.addupdate_scatter(deg_ref, [dst1], one, mask=lane == e)
        di = deg_ref[...].astype(jnp.int32)
        dinv = plsc.load_gather(fv, [_LUTR + (di >> 4), di & 15])

        norm0 = _gather16(dinv, src0) * _gather16(dinv, dst0)
        norm1 = _gather16(dinv, src1) * _gather16(dinv, dst1)
        for i in range(N):
            a_ref[i, :] = zero
        for e in range(16):
            plsc.addupdate_scatter(a_ref, [dst0, src0], norm0, mask=lane == e)
        for e in range(NE - 16):
            plsc.addupdate_scatter(a_ref, [dst1, src1], norm1, mask=lane == e)
        a = [[_bcast(a_ref[i, :], j) for j in range(N)] for i in range(N)]

        xv = [fv[_XR + n, :] for n in range(N)]
        w1 = [[fv[_W1R + k * 4 + b, :] for b in range(4)] for k in range(N)]
        t1 = []
        for n in range(N):
            acc = [None] * 4
            for k in range(N):
                s = _bcast(xv[n], k)
                for b in range(4):
                    p = s * w1[k][b]
                    acc[b] = p if acc[b] is None else acc[b] + p
            t1.append(acc)
        b1v = [fv[_B1R + b, :] for b in range(4)]
        h1 = []
        for i in range(N):
            hb = []
            for b in range(4):
                g = None
                for j in range(N):
                    p = a[i][j] * t1[j][b]
                    g = p if g is None else g + p
                hb.append(_bf16r(jnp.maximum(g + b1v[b], 0.0)))
            h1.append(hb)

        t2 = []
        for n in range(N):
            acc = [None, None]
            for k in range(64):
                s = _bcast(h1[n][k // 16], k % 16)
                for b in range(2):
                    p = s * fv[_W2R + k * 2 + b, :]
                    acc[b] = p if acc[b] is None else acc[b] + p
            t2.append(acc)
        b2v = [fv[_B2R + b, :] for b in range(2)]
        h2 = []
        for i in range(N):
            hb = []
            for b in range(2):
                g = None
                for j in range(N):
                    p = a[i][j] * t2[j][b]
                    g = p if g is None else g + p
                hb.append(_bf16r(jnp.maximum(g + b2v[b], 0.0)))
            h2.append(hb)

        w3v = [fv[_W3R, :], fv[_W3R + 1, :]]
        u = []
        for n in range(N):
            d = h2[n][0] * w3v[0] + h2[n][1] * w3v[1]
            u.append(_bcast(plsc.cumsum(d), 15))
        b3v = fv[_B3R, :]
        vsp = []
        for i in range(N):
            g = None
            for j in range(N):
                p = a[i][j] * u[j]
                g = p if g is None else g + p
            vsp.append(_bf16r(g + b3v))

        o = []
        for b in range(2):
            g = fv[_BLR + b, :]
            for i in range(N):
                g = g + vsp[i] * fv[_WLR + i * 2 + b, :]
            o.append(g)
        out_ref[0, :] = o[0]
        out_ref[1, :] = o[1]
        pltpu.sync_copy(out_ref, out_hbm)


def kernel(x, edge_index, W1, b1, W2, b2, W3, b3, Wl, bl):
    ei = edge_index.astype(jnp.int32)
    loop = jnp.arange(N, dtype=jnp.int32)
    src = jnp.pad(jnp.concatenate([ei[0], loop]), (0, 32 - NE)).reshape(2, 16)
    dst = jnp.pad(jnp.concatenate([ei[1], loop]), (0, 32 - NE)).reshape(2, 16)
    idx = jnp.concatenate([src, dst], axis=0)

    n = jnp.arange(32, dtype=jnp.float32)
    lut = jnp.where(n > 0, jnp.maximum(n, 1.0) ** -0.5, 0.0).reshape(2, 16)

    bfr = lambda a: lax.reduce_precision(a.astype(jnp.float32), 8, 7)

    pack = jnp.concatenate([
        jnp.pad(bfr(x.astype(jnp.float32)), ((0, 0), (0, 16 - x.shape[1]))),
        bfr(W1).reshape(20, 16),
        b1.reshape(4, 16),
        bfr(W2).reshape(128, 16),
        b2.reshape(2, 16),
        bfr(W3).reshape(-1).reshape(2, 16),
        jnp.broadcast_to(b3, (16,)).reshape(1, 16),
        jnp.pad(bfr(Wl).T, ((0, 0), (0, 7))).reshape(10, 16),
        jnp.pad(bl, (0, 7)).reshape(2, 16),
        lut,
    ], axis=0)

    res = _sc_gcn(pack, idx)
    return res.reshape(32)[:NE].reshape(N, N)


# speedup vs baseline: 1.1977x; 1.0331x over previous
import functools

import jax
import jax.numpy as jnp
from jax import lax
from jax.experimental import pallas as pl
from jax.experimental.pallas import tpu as pltpu
from jax.experimental.pallas import tpu_sc as plsc

N = 5
NE = 25

_XR = 0
_W1R = 5
_B1R = 25
_W2R = 29
_B2R = 157
_W3R = 159
_B3R = 161
_WLR = 162
_BLR = 172
_LUTR = 174
_ROWS = 176

_GDN = lax.GatherDimensionNumbers(
    offset_dims=(), collapsed_slice_dims=(0,), start_index_map=(0,))


def _gather16(v, idx):
    return lax.gather(v, idx[:, None], _GDN, (1,),
                      mode=lax.GatherScatterMode.PROMISE_IN_BOUNDS)


def _bcast(v, k):
    return _gather16(v, jnp.full((16,), k, dtype=jnp.int32))


def _bf16r(v):
    i = plsc.bitcast(v, jnp.int32)
    r = (i + jnp.int32(0x7FFF) + ((i >> 16) & 1)) & jnp.int32(-65536)
    return plsc.bitcast(r, jnp.float32)


@functools.partial(
    pl.kernel,
    out_type=jax.ShapeDtypeStruct((2, 16), jnp.float32),
    mesh=plsc.VectorSubcoreMesh(core_axis_name="c", subcore_axis_name="s",
                                num_cores=1),
    compiler_params=pltpu.CompilerParams(needs_layout_passes=False),
    scratch_types=[
        pltpu.VMEM((_ROWS, 16), jnp.float32),
        pltpu.VMEM((4, 16), jnp.int32),
        pltpu.VMEM((8, 16), jnp.float32),
        pltpu.VMEM((16,), jnp.float32),
        pltpu.VMEM((2, 16), jnp.float32),
    ],
)
def _sc_gcn(pack_hbm, idx_hbm, out_hbm, fv, iv, a_ref, deg_ref, out_ref):
    cid = lax.axis_index("c")
    sid = lax.axis_index("s")

    @pl.when(jnp.logical_and(cid == 0, sid == 0))
    def _body():
        pltpu.sync_copy(pack_hbm, fv)
        pltpu.sync_copy(idx_hbm, iv)

        lane = lax.iota(jnp.int32, 16)
        one = jnp.ones((16,), jnp.float32)
        zero = jnp.zeros((16,), jnp.float32)

        src0 = iv[0, :]
        src1 = iv[1, :]
        dst0 = iv[2, :]
        dst1 = iv[3, :]

        deg_ref[...] = zero
        for e in range(16):
            plsc.addupdate_scatter(deg_ref, [dst0], one, mask=lane == e)
        for e in range(NE - 16):
            plsc.addupdate_scatter(deg_ref, [dst1], one, mask=lane == e)
        di = deg_ref[...].astype(jnp.int32)
        dinv = plsc.load_gather(fv, [_LUTR + (di >> 4), di & 15])

        norm0 = _gather16(dinv, src0) * _gather16(dinv, dst0)
        norm1 = _gather16(dinv, src1) * _gather16(dinv, dst1)
        for i in range(N):
            a_ref[i, :] = zero
        for e in range(16):
            plsc.addupdate_scatter(a_ref, [dst0, src0], norm0, mask=lane == e)
        for e in range(NE - 16):
            plsc.addupdate_scatter(a_ref, [dst1, src1], norm1, mask=lane == e)
        a = [[_bcast(a_ref[i, :], j) for j in range(N)] for i in range(N)]

        xv = [fv[_XR + n, :] for n in range(N)]
        w1 = [[fv[_W1R + k * 4 + b, :] for b in range(4)] for k in range(N)]
        t1 = []
        for n in range(N):
            acc = [None] * 4
            for k in range(N):
                s = _bcast(xv[n], k)
                for b in range(4):
                    p = s * w1[k][b]
                    acc[b] = p if acc[b] is None else acc[b] + p
            t1.append(acc)
        b1v = [fv[_B1R + b, :] for b in range(4)]
        h1 = []
        for i in range(N):
            hb = []
            for b in range(4):
                g = None
                for j in range(N):
                    p = a[i][j] * t1[j][b]
                    g = p if g is None else g + p
                hb.append(_bf16r(jnp.maximum(g + b1v[b], 0.0)))
            h1.append(hb)

        t2 = []
        for n in range(N):
            acc = [None, None]
            for k in range(64):
                s = _bcast(h1[n][k // 16], k % 16)
                for b in range(2):
                    p = s * fv[_W2R + k * 2 + b, :]
                    acc[b] = p if acc[b] is None else acc[b] + p
            t2.append(acc)
        b2v = [fv[_B2R + b, :] for b in range(2)]
        h2 = []
        for i in range(N):
            hb = []
            for b in range(2):
                g = None
                for j in range(N):
                    p = a[i][j] * t2[j][b]
                    g = p if g is None else g + p
                hb.append(_bf16r(jnp.maximum(g + b2v[b], 0.0)))
            h2.append(hb)

        w3v = [fv[_W3R, :], fv[_W3R + 1, :]]
        u = []
        for n in range(N):
            d = h2[n][0] * w3v[0] + h2[n][1] * w3v[1]
            u.append(_bcast(plsc.cumsum(d), 15))
        b3v = fv[_B3R, :]
        vsp = []
        for i in range(N):
            g = None
            for j in range(N):
                p = a[i][j] * u[j]
                g = p if g is None else g + p
            vsp.append(_bf16r(g + b3v))

        o = []
        for b in range(2):
            g = fv[_BLR + b, :]
            for i in range(N):
                g = g + vsp[i] * fv[_WLR + i * 2 + b, :]
            o.append(g)
        out_ref[0, :] = o[0]
        out_ref[1, :] = o[1]
        pltpu.sync_copy(out_ref, out_hbm)


def kernel(x, edge_index, W1, b1, W2, b2, W3, b3, Wl, bl):
    ei = edge_index.astype(jnp.int32)
    loop = jnp.arange(N, dtype=jnp.int32)
    src = jnp.pad(jnp.concatenate([ei[0], loop]), (0, 32 - NE)).reshape(2, 16)
    dst = jnp.pad(jnp.concatenate([ei[1], loop]), (0, 32 - NE)).reshape(2, 16)
    idx = jnp.concatenate([src, dst], axis=0)

    n = jnp.arange(32, dtype=jnp.float32)
    lut = jnp.where(n > 0, jnp.maximum(n, 1.0) ** -0.5, 0.0).reshape(2, 16)

    bfr = lambda a: lax.reduce_precision(a.astype(jnp.float32), 8, 7)

    pack = jnp.concatenate([
        jnp.pad(bfr(x.astype(jnp.float32)), ((0, 0), (0, 16 - x.shape[1]))),
        bfr(W1).reshape(20, 16),
        b1.reshape(4, 16),
        bfr(W2).reshape(128, 16),
        b2.reshape(2, 16),
        bfr(W3).reshape(-1).reshape(2, 16),
        jnp.broadcast_to(b3, (16,)).reshape(1, 16),
        jnp.pad(bfr(Wl).T, ((0, 0), (0, 7))).reshape(10, 16),
        jnp.pad(bl, (0, 7)).reshape(2, 16),
        lut,
    ], axis=0)

    res = _sc_gcn(pack, idx)
    return res.reshape(32)[:NE].reshape(N, N)


# speedup vs baseline: 1.6758x; 1.3992x over previous
import functools

import jax
import jax.numpy as jnp
from jax import lax
from jax.experimental import pallas as pl
from jax.experimental.pallas import tpu as pltpu
from jax.experimental.pallas import tpu_sc as plsc

N = 5
NE = 25

_GDN = lax.GatherDimensionNumbers(
    offset_dims=(), collapsed_slice_dims=(0,), start_index_map=(0,))


def _gather16(v, idx):
    return lax.gather(v, idx[:, None], _GDN, (1,),
                      mode=lax.GatherScatterMode.PROMISE_IN_BOUNDS)


def _bcast(v, k):
    return _gather16(v, jnp.full((16,), k, dtype=jnp.int32))


def _bf16r(v):
    i = plsc.bitcast(v, jnp.int32)
    r = (i + jnp.int32(0x7FFF) + ((i >> 16) & 1)) & jnp.int32(-65536)
    return plsc.bitcast(r, jnp.float32)


def _rsqrt16(x):
    i = plsc.bitcast(x, jnp.int32)
    y = plsc.bitcast(jnp.int32(0x5F3759DF) - (i >> 1), jnp.float32)
    for _ in range(3):
        y = y * (1.5 - 0.5 * x * y * y)
    return y


@functools.partial(
    pl.kernel,
    out_type=jax.ShapeDtypeStruct((2, 16), jnp.float32),
    mesh=plsc.VectorSubcoreMesh(core_axis_name="c", subcore_axis_name="s",
                                num_cores=1),
    compiler_params=pltpu.CompilerParams(needs_layout_passes=False),
    scratch_types=[
        pltpu.VMEM((2, 16), jnp.float32),
        pltpu.VMEM((2, 32), jnp.int32),
        pltpu.VMEM((N, 64), jnp.float32),
        pltpu.VMEM((64,), jnp.float32),
        pltpu.VMEM((64, 32), jnp.float32),
        pltpu.VMEM((32,), jnp.float32),
        pltpu.VMEM((32,), jnp.float32),
        pltpu.VMEM((16,), jnp.float32),
        pltpu.VMEM((N, 32), jnp.float32),
        pltpu.VMEM((2, 16), jnp.float32),
        pltpu.VMEM((8, 16), jnp.float32),
        pltpu.VMEM((16,), jnp.float32),
        pltpu.VMEM((2, 16), jnp.float32),
    ] + [pltpu.SemaphoreType.DMA] * 10,
)
def _sc_gcn(x_h, ei_h, w1_h, b1_h, w2_h, b2_h, w3_h, b3_h, wl_h, bl_h,
            out_hbm, xr, eir, w1r, b1r, w2r, b2r, w3r, b3r, wlr, blr,
            a_ref, deg_ref, out_ref, *sems):
    cid = lax.axis_index("c")
    sid = lax.axis_index("s")

    @pl.when(jnp.logical_and(cid == 0, sid == 0))
    def _body():
        srcs = [x_h, ei_h, w1_h, b1_h, w2_h, b2_h, w3_h, b3_h, wl_h, bl_h]
        dsts = [xr, eir, w1r, b1r, w2r, b2r, w3r, b3r, wlr, blr]
        copies = [pltpu.make_async_copy(s, d, sm)
                  for s, d, sm in zip(srcs, dsts, sems)]
        for c in copies:
            c.start()
        for c in copies:
            c.wait()

        lane = lax.iota(jnp.int32, 16)
        one = jnp.ones((16,), jnp.float32)
        zero = jnp.zeros((16,), jnp.float32)

        src0 = eir[0, 0:16]
        dst0 = eir[1, 0:16]
        src1 = jnp.where(lane < 4, eir[0, 16:32], lane - 4)
        dst1 = jnp.where(lane < 4, eir[1, 16:32], lane - 4)

        deg_ref[...] = zero
        for e in range(16):
            plsc.addupdate_scatter(deg_ref, [dst0], one, mask=lane == e)
        for e in range(NE - 16):
            plsc.addupdate_scatter(deg_ref, [dst1], one, mask=lane == e)
        dinv = _rsqrt16(jnp.maximum(deg_ref[...], 1.0))

        norm0 = _gather16(dinv, src0) * _gather16(dinv, dst0)
        norm1 = _gather16(dinv, src1) * _gather16(dinv, dst1)
        for i in range(N):
            a_ref[i, :] = zero
        for e in range(16):
            plsc.addupdate_scatter(a_ref, [dst0, src0], norm0, mask=lane == e)
        for e in range(NE - 16):
            plsc.addupdate_scatter(a_ref, [dst1, src1], norm1, mask=lane == e)
        a = [[_bcast(a_ref[i, :], j) for j in range(N)] for i in range(N)]

        xv = [_bf16r(xr[i, :]) for i in range(2)]
        t1 = [[None] * 4 for _ in range(N)]
        for k in range(N):
            w1k = [_bf16r(w1r[k, 16 * b:16 * (b + 1)]) for b in range(4)]
            for n in range(N):
                f = 5 * n + k
                s = _bcast(xv[f // 16], f % 16)
                for b in range(4):
                    p = s * w1k[b]
                    t1[n][b] = p if t1[n][b] is None else t1[n][b] + p
        b1v = [b1r[pl.ds(16 * b, 16)] for b in range(4)]
        h1 = []
        for i in range(N):
            hb = []
            for b in range(4):
                g = None
                for j in range(N):
                    p = a[i][j] * t1[j][b]
                    g = p if g is None else g + p
                hb.append(_bf16r(jnp.maximum(g + b1v[b], 0.0)))
            h1.append(hb)

        t2 = [[None, None] for _ in range(N)]
        for k in range(64):
            w2k = [_bf16r(w2r[k, 0:16]), _bf16r(w2r[k, 16:32])]
            for n in range(N):
                s = _bcast(h1[n][k // 16], k % 16)
                for b in range(2):
                    p = s * w2k[b]
                    t2[n][b] = p if t2[n][b] is None else t2[n][b] + p
        b2v = [b2r[pl.ds(16 * b, 16)] for b in range(2)]
        h2 = []
        for i in range(N):
            hb = []
            for b in range(2):
                g = None
                for j in range(N):
                    p = a[i][j] * t2[j][b]
                    g = p if g is None else g + p
                hb.append(_bf16r(jnp.maximum(g + b2v[b], 0.0)))
            h2.append(hb)

        w3v = [_bf16r(w3r[pl.ds(16 * b, 16)]) for b in range(2)]
        u = []
        for n in range(N):
            d = h2[n][0] * w3v[0] + h2[n][1] * w3v[1]
            u.append(_bcast(plsc.cumsum(d), 15))
        b3v = b3r[...]
        vsp = []
        for i in range(N):
            g = None
            for j in range(N):
                p = a[i][j] * u[j]
                g = p if g is None else g + p
            vsp.append(_bf16r(g + b3v))

        o = []
        for b in range(2):
            g = blr[b, :]
            for i in range(N):
                g = g + vsp[i] * _bf16r(wlr[i, 16 * b:16 * (b + 1)])
            o.append(g)
        out_ref[0, :] = o[0]
        out_ref[1, :] = o[1]
        pltpu.sync_copy(out_ref, out_hbm)


def kernel(x, edge_index, W1, b1, W2, b2, W3, b3, Wl, bl):
    xp = jnp.pad(x.reshape(NE), (0, 7)).reshape(2, 16)
    eip = jnp.pad(edge_index.astype(jnp.int32), ((0, 0), (0, 12)))
    wlp = jnp.pad(Wl.T, ((0, 0), (0, 7)))
    blp = jnp.pad(bl, (0, 7)).reshape(2, 16)
    res = _sc_gcn(xp, eip, W1, b1, W2, b2, W3.reshape(32),
                  jnp.broadcast_to(b3, (16,)), wlp, blp)
    return res.reshape(32)[:NE].reshape(N, N)


# speedup vs baseline: 1.8492x; 1.1035x over previous
import functools

import jax
import jax.numpy as jnp
from jax import lax
from jax.experimental import pallas as pl
from jax.experimental.pallas import tpu as pltpu
from jax.experimental.pallas import tpu_sc as plsc

N = 5
NE = 25

_GDN = lax.GatherDimensionNumbers(
    offset_dims=(), collapsed_slice_dims=(0,), start_index_map=(0,))


def _gather16(v, idx):
    return lax.gather(v, idx[:, None], _GDN, (1,),
                      mode=lax.GatherScatterMode.PROMISE_IN_BOUNDS)


def _bcast(v, k):
    return _gather16(v, jnp.full((16,), k, dtype=jnp.int32))


def _bf16r(v):
    i = plsc.bitcast(v, jnp.int32)
    r = (i + jnp.int32(0x7FFF) + ((i >> 16) & 1)) & jnp.int32(-65536)
    return plsc.bitcast(r, jnp.float32)


def _rsqrt16(x):
    i = plsc.bitcast(x, jnp.int32)
    y = plsc.bitcast(jnp.int32(0x5F3759DF) - (i >> 1), jnp.float32)
    for _ in range(3):
        y = y * (1.5 - 0.5 * x * y * y)
    return y


@functools.partial(
    pl.kernel,
    out_type=jax.ShapeDtypeStruct((N, N), jnp.float32),
    mesh=plsc.VectorSubcoreMesh(core_axis_name="c", subcore_axis_name="s",
                                num_cores=1),
    compiler_params=pltpu.CompilerParams(needs_layout_passes=False),
    scratch_types=[
        pltpu.VMEM((N, 16), jnp.float32),
        pltpu.VMEM((2, 32), jnp.int32),
        pltpu.VMEM((N, 64), jnp.float32),
        pltpu.VMEM((64,), jnp.float32),
        pltpu.VMEM((64, 32), jnp.float32),
        pltpu.VMEM((32,), jnp.float32),
        pltpu.VMEM((32,), jnp.float32),
        pltpu.VMEM((16,), jnp.float32),
        pltpu.VMEM((N, 32), jnp.float32),
        pltpu.VMEM((32,), jnp.float32),
        pltpu.VMEM((8, 16), jnp.float32),
        pltpu.VMEM((16,), jnp.float32),
        pltpu.VMEM((N, N), jnp.float32),
    ] + [pltpu.SemaphoreType.DMA] * 10,
)
def _sc_gcn(x_h, ei_h, w1_h, b1_h, w2_h, b2_h, w3_h, b3_h, wl_h, bl_h,
            out_hbm, xr, eir, w1r, b1r, w2r, b2r, w3r, b3r, wlr, blr,
            a_ref, deg_ref, out_ref, *sems):
    cid = lax.axis_index("c")
    sid = lax.axis_index("s")

    @pl.when(jnp.logical_and(cid == 0, sid == 0))
    def _body():
        srcs = [x_h, ei_h, w1_h, b1_h, w2_h, b2_h, w3_h, b3_h, wl_h, bl_h]
        dsts = [xr, eir, w1r, b1r, w2r, b2r, w3r, b3r, wlr, blr]
        copies = [pltpu.make_async_copy(s, d, sm)
                  for s, d, sm in zip(srcs, dsts, sems)]
        for c in copies:
            c.start()
        for c in copies:
            c.wait()

        lane = lax.iota(jnp.int32, 16)
        one = jnp.ones((16,), jnp.float32)
        zero = jnp.zeros((16,), jnp.float32)

        src0 = eir[0, 0:16]
        dst0 = eir[1, 0:16]
        src1 = jnp.where(lane < 4, eir[0, 16:32], lane - 4)
        dst1 = jnp.where(lane < 4, eir[1, 16:32], lane - 4)

        deg_ref[...] = zero
        for e in range(16):
            plsc.addupdate_scatter(deg_ref, [dst0], one, mask=lane == e)
        for e in range(NE - 16):
            plsc.addupdate_scatter(deg_ref, [dst1], one, mask=lane == e)
        dinv = _rsqrt16(jnp.maximum(deg_ref[...], 1.0))

        norm0 = _gather16(dinv, src0) * _gather16(dinv, dst0)
        norm1 = _gather16(dinv, src1) * _gather16(dinv, dst1)
        for i in range(N):
            a_ref[i, :] = zero
        for e in range(16):
            plsc.addupdate_scatter(a_ref, [dst0, src0], norm0, mask=lane == e)
        for e in range(NE - 16):
            plsc.addupdate_scatter(a_ref, [dst1, src1], norm1, mask=lane == e)
        a = [[_bcast(a_ref[i, :], j) for j in range(N)] for i in range(N)]

        xv = [_bf16r(xr[n, :]) for n in range(N)]
        t1 = [[None] * 4 for _ in range(N)]
        for k in range(N):
            w1k = [_bf16r(w1r[k, 16 * b:16 * (b + 1)]) for b in range(4)]
            for n in range(N):
                s = _bcast(xv[n], k)
                for b in range(4):
                    p = s * w1k[b]
                    t1[n][b] = p if t1[n][b] is None else t1[n][b] + p
        b1v = [b1r[pl.ds(16 * b, 16)] for b in range(4)]
        h1 = []
        for i in range(N):
            hb = []
            for b in range(4):
                g = None
                for j in range(N):
                    p = a[i][j] * t1[j][b]
                    g = p if g is None else g + p
                hb.append(_bf16r(jnp.maximum(g + b1v[b], 0.0)))
            h1.append(hb)

        t2 = [[None, None] for _ in range(N)]
        for k in range(64):
            w2k = [_bf16r(w2r[k, 0:16]), _bf16r(w2r[k, 16:32])]
            for n in range(N):
                s = _bcast(h1[n][k // 16], k % 16)
                for b in range(2):
                    p = s * w2k[b]
                    t2[n][b] = p if t2[n][b] is None else t2[n][b] + p
        b2v = [b2r[pl.ds(16 * b, 16)] for b in range(2)]
        h2 = []
        for i in range(N):
            hb = []
            for b in range(2):
                g = None
                for j in range(N):
                    p = a[i][j] * t2[j][b]
                    g = p if g is None else g + p
                hb.append(_bf16r(jnp.maximum(g + b2v[b], 0.0)))
            h2.append(hb)

        w3v = [_bf16r(w3r[pl.ds(16 * b, 16)]) for b in range(2)]
        u = []
        for n in range(N):
            d = h2[n][0] * w3v[0] + h2[n][1] * w3v[1]
            u.append(_bcast(plsc.cumsum(d), 15))
        b3v = b3r[...]
        vsp = []
        for i in range(N):
            g = None
            for j in range(N):
                p = a[i][j] * u[j]
                g = p if g is None else g + p
            vsp.append(_bf16r(g + b3v))

        for b, nval in ((0, 16), (1, NE - 16)):
            g = blr[pl.ds(16 * b, 16)]
            for i in range(N):
                g = g + vsp[i] * _bf16r(wlr[i, 16 * b:16 * (b + 1)])
            m = lane + 16 * b
            rows = (m * 52429) >> 18
            cols = m - N * rows
            plsc.store_scatter(out_ref, [rows, cols], g, mask=lane < nval)
        pltpu.sync_copy(out_ref, out_hbm)


def kernel(x, edge_index, W1, b1, W2, b2, W3, b3, Wl, bl):
    xp = jnp.pad(x, ((0, 0), (0, 16 - x.shape[1])))
    eip = jnp.pad(edge_index.astype(jnp.int32), ((0, 0), (0, 12)))
    wlp = jnp.pad(Wl.T, ((0, 0), (0, 7)))
    blp = jnp.pad(bl, (0, 7))
    return _sc_gcn(xp, eip, W1, b1, W2, b2, W3.reshape(32),
                   jnp.broadcast_to(b3, (16,)), wlp, blp)


# speedup vs baseline: 1.8741x; 1.0135x over previous
import functools

import jax
import jax.numpy as jnp
from jax import lax
from jax.experimental import pallas as pl
from jax.experimental.pallas import tpu as pltpu
from jax.experimental.pallas import tpu_sc as plsc

N = 5
NE = 25

_GDN = lax.GatherDimensionNumbers(
    offset_dims=(), collapsed_slice_dims=(0,), start_index_map=(0,))


def _gather16(v, idx):
    return lax.gather(v, idx[:, None], _GDN, (1,),
                      mode=lax.GatherScatterMode.PROMISE_IN_BOUNDS)


def _bcast(v, k):
    return _gather16(v, jnp.full((16,), k, dtype=jnp.int32))


def _bf16r(v):
    i = plsc.bitcast(v, jnp.int32)
    r = (i + jnp.int32(0x7FFF) + ((i >> 16) & 1)) & jnp.int32(-65536)
    return plsc.bitcast(r, jnp.float32)


def _rsqrt16(x):
    i = plsc.bitcast(x, jnp.int32)
    y = plsc.bitcast(jnp.int32(0x5F3759DF) - (i >> 1), jnp.float32)
    for _ in range(3):
        y = y * (1.5 - 0.5 * x * y * y)
    return y


@functools.partial(
    pl.kernel,
    out_type=jax.ShapeDtypeStruct((N, N), jnp.float32),
    mesh=plsc.VectorSubcoreMesh(core_axis_name="c", subcore_axis_name="s",
                                num_cores=1),
    compiler_params=pltpu.CompilerParams(needs_layout_passes=False),
    scratch_types=[
        pltpu.VMEM((N, 16), jnp.float32),
        pltpu.VMEM((2, 32), jnp.int32),
        pltpu.VMEM((N, 64), jnp.float32),
        pltpu.VMEM((64,), jnp.float32),
        pltpu.VMEM((64, 32), jnp.float32),
        pltpu.VMEM((32,), jnp.float32),
        pltpu.VMEM((32,), jnp.float32),
        pltpu.VMEM((16,), jnp.float32),
        pltpu.VMEM((N, 32), jnp.float32),
        pltpu.VMEM((32,), jnp.float32),
        pltpu.VMEM((8, 16), jnp.float32),
        pltpu.VMEM((16,), jnp.float32),
        pltpu.VMEM((N, N), jnp.float32),
    ] + [pltpu.SemaphoreType.DMA] * 10,
)
def _sc_gcn(x_h, ei_h, w1_h, b1_h, w2_h, b2_h, w3_h, b3_h, wl_h, bl_h,
            out_hbm, xr, eir, w1r, b1r, w2r, b2r, w3r, b3r, wlr, blr,
            a_ref, deg_ref, out_ref, *sems):
    cid = lax.axis_index("c")
    sid = lax.axis_index("s")

    @pl.when(jnp.logical_and(cid == 0, sid == 0))
    def _body():
        srcs = [x_h, ei_h, w1_h, b1_h, w2_h, b2_h, w3_h, b3_h, wl_h, bl_h]
        dsts = [xr, eir, w1r, b1r, w2r, b2r, w3r, b3r, wlr,
                blr.at[pl.ds(0, NE)]]
        copies = [pltpu.make_async_copy(s, d, sm)
                  for s, d, sm in zip(srcs, dsts, sems)]
        for c in copies:
            c.start()
        for c in copies:
            c.wait()

        lane = lax.iota(jnp.int32, 16)
        one = jnp.ones((16,), jnp.float32)
        zero = jnp.zeros((16,), jnp.float32)

        src0 = eir[0, 0:16]
        dst0 = eir[1, 0:16]
        src1 = jnp.where(lane < 4, eir[0, 16:32], lane - 4)
        dst1 = jnp.where(lane < 4, eir[1, 16:32], lane - 4)

        deg_ref[...] = zero
        for e in range(16):
            plsc.addupdate_scatter(deg_ref, [dst0], one, mask=lane == e)
        for e in range(NE - 16):
            plsc.addupdate_scatter(deg_ref, [dst1], one, mask=lane == e)
        dinv = _rsqrt16(jnp.maximum(deg_ref[...], 1.0))

        norm0 = _gather16(dinv, src0) * _gather16(dinv, dst0)
        norm1 = _gather16(dinv, src1) * _gather16(dinv, dst1)
        for i in range(N):
            a_ref[i, :] = zero
        for e in range(16):
            plsc.addupdate_scatter(a_ref, [dst0, src0], norm0, mask=lane == e)
        for e in range(NE - 16):
            plsc.addupdate_scatter(a_ref, [dst1, src1], norm1, mask=lane == e)
        a = [[_bcast(a_ref[i, :], j) for j in range(N)] for i in range(N)]

        xv = [_bf16r(xr[n, :]) for n in range(N)]
        t1 = [[None] * 4 for _ in range(N)]
        for k in range(N):
            w1k = [_bf16r(w1r[k, 16 * b:16 * (b + 1)]) for b in range(4)]
            for n in range(N):
                s = _bcast(xv[n], k)
                for b in range(4):
                    p = s * w1k[b]
                    t1[n][b] = p if t1[n][b] is None else t1[n][b] + p
        b1v = [b1r[pl.ds(16 * b, 16)] for b in range(4)]
        h1 = []
        for i in range(N):
            hb = []
            for b in range(4):
                g = None
                for j in range(N):
                    p = a[i][j] * t1[j][b]
                    g = p if g is None else g + p
                hb.append(_bf16r(jnp.maximum(g + b1v[b], 0.0)))
            h1.append(hb)

        t2 = [[None, None] for _ in range(N)]
        for k in range(64):
            w2k = [_bf16r(w2r[k, 0:16]), _bf16r(w2r[k, 16:32])]
            for n in range(N):
                s = _bcast(h1[n][k // 16], k % 16)
                for b in range(2):
                    p = s * w2k[b]
                    t2[n][b] = p if t2[n][b] is None else t2[n][b] + p
        b2v = [b2r[pl.ds(16 * b, 16)] for b in range(2)]
        h2 = []
        for i in range(N):
            hb = []
            for b in range(2):
                g = None
                for j in range(N):
                    p = a[i][j] * t2[j][b]
                    g = p if g is None else g + p
                hb.append(_bf16r(jnp.maximum(g + b2v[b], 0.0)))
            h2.append(hb)

        w3v = [_bf16r(w3r[pl.ds(16 * b, 16)]) for b in range(2)]
        u = []
        for n in range(N):
            d = h2[n][0] * w3v[0] + h2[n][1] * w3v[1]
            u.append(_bcast(plsc.cumsum(d), 15))
        b3v = b3r[...]
        vsp = []
        for i in range(N):
            g = None
            for j in range(N):
                p = a[i][j] * u[j]
                g = p if g is None else g + p
            vsp.append(_bf16r(g + b3v))

        for b, nval in ((0, 16), (1, NE - 16)):
            g = blr[pl.ds(16 * b, 16)]
            for i in range(N):
                g = g + vsp[i] * _bf16r(wlr[i, 16 * b:16 * (b + 1)])
            m = lane + 16 * b
            rows = (m * 52429) >> 18
            cols = m - N * rows
            plsc.store_scatter(out_ref, [rows, cols], g, mask=lane < nval)
        pltpu.sync_copy(out_ref, out_hbm)


def kernel(x, edge_index, W1, b1, W2, b2, W3, b3, Wl, bl):
    xp = jnp.pad(x, ((0, 0), (0, 16 - x.shape[1])))
    eip = jnp.pad(edge_index.astype(jnp.int32), ((0, 0), (0, 12)))
    wlp = jnp.pad(Wl.T, ((0, 0), (0, 7)))
    return _sc_gcn(xp, eip, W1, b1, W2, b2, W3.reshape(32),
                   jnp.broadcast_to(b3, (16,)), wlp, bl)
